# baseline (speedup 1.0000x reference)
import functools

import jax
import jax.numpy as jnp
from jax import lax
from jax.experimental import pallas as pl
from jax.experimental.pallas import tpu as pltpu
from jax.experimental.pallas import tpu_sc as plsc

VOCAB = 1000000
EMBED = 64
CTX = 20
BATCH = 16384

NW = 32
PER_W = BATCH // NW
CH = 8
NCHUNK = PER_W // CH
CROWS = CH * CTX
L = 16
W128 = 2 * EMBED
CPW = PER_W * CTX


def _sc_body(b_hbm, cf_hbm, nb_hbm, ncf_hbm, emb_hbm, ctx_hbm, simi_hbm,
             out_hbm, braw, nbraw, craw, ncraw,
             bidx_a, nbidx_a, cidx_a, ncidx_a,
             bidx_b, nbidx_b, cidx_b, ncidx_b,
             be_a, nbe_a, ce_a, nce_a, be_b, nbe_b, ce_b, nce_b,
             ctxv, simiv, outv, wbuf, sem_a, sem_b):
    cid = lax.axis_index("c")
    sid = lax.axis_index("s")
    wid = sid * 2 + cid
    base = wid * PER_W
    lanes = lax.iota(jnp.int32, L)

    ob = pl.multiple_of(base, PER_W)
    o20 = pl.multiple_of(base * CTX, CPW)
    pltpu.sync_copy(b_hbm.at[pl.ds(ob, PER_W)], braw.at[pl.ds(0, PER_W)])
    pltpu.sync_copy(nb_hbm.at[pl.ds(ob, PER_W)], nbraw.at[pl.ds(0, PER_W)])
    pltpu.sync_copy(cf_hbm.at[pl.ds(o20, CPW)], craw)
    pltpu.sync_copy(ncf_hbm.at[pl.ds(o20, CPW)], ncraw)
    braw[pl.ds(PER_W, L)] = jnp.zeros((L,), jnp.int32)
    nbraw[pl.ds(PER_W, L)] = jnp.zeros((L,), jnp.int32)

    pltpu.sync_copy(ctx_hbm, ctxv)
    x0 = ctxv[pl.ds(0, L)]
    x1 = ctxv[pl.ds(L, L)]

    _dnums = lax.GatherDimensionNumbers(
        offset_dims=(), collapsed_slice_dims=(0,), start_index_map=(0,))

    def _gat(x, idx):
        return lax.gather(x, idx[:, None], _dnums, (1,),
                          mode=lax.GatherScatterMode.PROMISE_IN_BOUNDS)

    def _rot(x, sh):
        return _gat(x, (lanes + sh) & (L - 1))

    def _splat(x, j):
        return _gat(x, jnp.full((L,), j, jnp.int32))

    m = jnp.maximum(x0, x1)
    for sh in (8, 4, 2, 1):
        m = jnp.maximum(m, _rot(m, sh))
    e0 = jnp.exp(x0 - m)
    e1 = jnp.exp(x1 - m)
    s = e0 + e1
    for sh in (8, 4, 2, 1):
        s = s + _rot(s, sh)
    inv_s = 1.0 / s
    for j in range(CTX):
        src = e0 if j < L else e1
        wbuf[pl.ds(j * L, L)] = _splat(src, j % L) * inv_s

    tl = [t * L + lanes for t in range(4)]

    bufs = ((bidx_a, nbidx_a, cidx_a, ncidx_a, be_a, nbe_a, ce_a, nce_a,
             sem_a),
            (bidx_b, nbidx_b, cidx_b, ncidx_b, be_b, nbe_b, ce_b, nce_b,
             sem_b))

    def issue(c, which):
        bidx, nbidx, cidx, ncidx, be, nbe, ce, nce, sem = bufs[which]
        coff = c * CROWS
        boff = c * CH
        for g in range(CTX // 2):
            gs = pl.ds(g * L, L)
            cidx[gs] = craw[pl.ds(coff + g * L, L)] >> 1
            ncidx[gs] = ncraw[pl.ds(coff + g * L, L)] >> 1
        bv = braw[pl.ds(boff, L)] >> 1
        nbv = nbraw[pl.ds(boff, L)] >> 1
        bidx[...] = bv
        nbidx[...] = nbv
        for lo, n in ((0, 128), (128, 32)):
            pltpu.async_copy(
                emb_hbm.at[cidx.at[pl.ds(lo, n)]], ce.at[pl.ds(lo, n)], sem)
            pltpu.async_copy(
                emb_hbm.at[ncidx.at[pl.ds(lo, n)]], nce.at[pl.ds(lo, n)],
                sem)
        pltpu.async_copy(emb_hbm.at[bidx], be, sem)
        pltpu.async_copy(emb_hbm.at[nbidx], nbe, sem)

    def drain(which):
        bidx, nbidx, cidx, ncidx, be, nbe, ce, nce, sem = bufs[which]
        pltpu.make_async_copy(emb_hbm.at[pl.ds(0, CROWS)], ce, sem).wait()
        pltpu.make_async_copy(emb_hbm.at[pl.ds(0, CROWS)], nce, sem).wait()
        pltpu.make_async_copy(emb_hbm.at[pl.ds(0, L)], be, sem).wait()
        pltpu.make_async_copy(emb_hbm.at[pl.ds(0, L)], nbe, sem).wait()

    def compute(c, which, acc):
        _, _, _, _, be, nbe, ce, nce, _ = bufs[which]
        coff = c * CROWS
        boff = c * CH
        bpo = (braw[pl.ds(boff, L)] & 1) * EMBED
        nbpo = (nbraw[pl.ds(boff, L)] & 1) * EMBED

        @plsc.parallel_loop(0, CH, carry=acc)
        def elem(i, el_acc):
            a0, a1, a2, a3 = el_acc
            r0 = i * CTX
            iv = jnp.full((L,), i, jnp.int32)
            bcol = _gat(bpo, iv)
            nbcol = _gat(nbpo, iv)
            bs = [plsc.load_gather(be, [iv, bcol + tl[t]]) for t in range(4)]
            ns = [plsc.load_gather(nbe, [iv, nbcol + tl[t]])
                  for t in range(4)]
            u = [0.5 * ns[t] - bs[t] for t in range(4)]
            v = [0.5 * bs[t] for t in range(4)]
            ra = craw[pl.ds(coff + r0, L)]
            rb = craw[pl.ds(coff + r0 + 4, L)]
            nra = ncraw[pl.ds(coff + r0, L)]
            nrb = ncraw[pl.ds(coff + r0 + 4, L)]
            fba = (ra & 1) * EMBED
            fbb = (rb & 1) * EMBED
            nfba = (nra & 1) * EMBED
            nfbb = (nrb & 1) * EMBED
            for j in range(CTX):
                r = r0 + j
                rrow = jnp.full((L,), r, jnp.int32)
                if j < L:
                    ccol = _splat(fba, j)
                    nccol = _splat(nfba, j)
                else:
                    ccol = _splat(fbb, j - 4)
                    nccol = _splat(nfbb, j - 4)
                cs = [plsc.load_gather(ce, [rrow, ccol + tl[t]])
                      for t in range(4)]
                nc = [plsc.load_gather(nce, [rrow, nccol + tl[t]])
                      for t in range(4)]
                t0 = ((u[0] * cs[0] + u[1] * cs[1])
                      + (u[2] * cs[2] + u[3] * cs[3]))
                t1 = ((v[0] * nc[0] + v[1] * nc[1])
                      + (v[2] * nc[2] + v[3] * nc[3]))
                wv = wbuf[pl.ds(j * L, L)]
                term = wv * (t0 + t1)
                if j % 4 == 0:
                    a0 = a0 + term
                elif j % 4 == 1:
                    a1 = a1 + term
                elif j % 4 == 2:
                    a2 = a2 + term
                else:
                    a3 = a3 + term
            return (a0, a1, a2, a3)

        return elem

    issue(0, 0)
    issue(1, 1)

    def pair(mm, acc):
        c0 = mm * 2
        drain(0)
        acc = compute(c0, 0, acc)

        @pl.when(mm < NCHUNK // 2 - 1)
        def _():
            issue(c0 + 2, 0)

        drain(1)
        acc = compute(c0 + 1, 1, acc)

        @pl.when(mm < NCHUNK // 2 - 1)
        def _():
            issue(c0 + 3, 1)

        return acc

    z = jnp.zeros((L,), jnp.float32)
    a0, a1, a2, a3 = lax.fori_loop(0, NCHUNK // 2, pair, (z, z, z, z))
    outv[...] = (a0 + a1) + (a2 + a3)

    @pl.when(wid == 0)
    def _():
        pltpu.sync_copy(simi_hbm, simiv)

        def sbody(k, a2s):
            off = pl.multiple_of(k * L, L)
            vv = simiv[pl.ds(off, L)]
            return a2s + vv * vv

        a2s = lax.fori_loop(0, (EMBED * EMBED) // L, sbody,
                            jnp.zeros((L,), jnp.float32))
        outv[...] = outv[...] + (0.1 * BATCH / (EMBED * EMBED)) * a2s

    oo = pl.multiple_of(wid * L, L)
    pltpu.sync_copy(outv, out_hbm.at[pl.ds(oo, L)])


@functools.partial(
    pl.kernel,
    out_type=jax.ShapeDtypeStruct((NW * L,), jnp.float32),
    mesh=plsc.VectorSubcoreMesh(core_axis_name="c", subcore_axis_name="s"),
    compiler_params=pltpu.CompilerParams(needs_layout_passes=False),
    scratch_types=[
        pltpu.VMEM((PER_W + L,), jnp.int32),
        pltpu.VMEM((PER_W + L,), jnp.int32),
        pltpu.VMEM((CPW,), jnp.int32),
        pltpu.VMEM((CPW,), jnp.int32),
        pltpu.VMEM((L,), jnp.int32),
        pltpu.VMEM((L,), jnp.int32),
        pltpu.VMEM((CROWS,), jnp.int32),
        pltpu.VMEM((CROWS,), jnp.int32),
        pltpu.VMEM((L,), jnp.int32),
        pltpu.VMEM((L,), jnp.int32),
        pltpu.VMEM((CROWS,), jnp.int32),
        pltpu.VMEM((CROWS,), jnp.int32),
        pltpu.VMEM((L, W128), jnp.float32),
        pltpu.VMEM((L, W128), jnp.float32),
        pltpu.VMEM((CROWS, W128), jnp.float32),
        pltpu.VMEM((CROWS, W128), jnp.float32),
        pltpu.VMEM((L, W128), jnp.float32),
        pltpu.VMEM((L, W128), jnp.float32),
        pltpu.VMEM((CROWS, W128), jnp.float32),
        pltpu.VMEM((CROWS, W128), jnp.float32),
        pltpu.VMEM((32,), jnp.float32),
        pltpu.VMEM((EMBED * EMBED,), jnp.float32),
        pltpu.VMEM((L,), jnp.float32),
        pltpu.VMEM((CTX * L,), jnp.float32),
        pltpu.SemaphoreType.DMA,
        pltpu.SemaphoreType.DMA,
    ],
)
def _sc_entry(*refs):
    _sc_body(*refs)


def kernel(b, C, nb, nC, emb_weight, ctx_scheme, simi_kernel):
    b32 = b.astype(jnp.int32)
    nb32 = nb.astype(jnp.int32)
    cf = C.reshape(-1).astype(jnp.int32)
    ncf = nC.reshape(-1).astype(jnp.int32)
    emb2 = emb_weight.reshape(VOCAB // 2, W128)
    ctx_pad = jnp.concatenate(
        [ctx_scheme.astype(jnp.float32),
         jnp.full((32 - CTX,), -1e30, jnp.float32)])
    partials = _sc_entry(b32, cf, nb32, ncf, emb2, ctx_pad,
                         simi_kernel.reshape(-1))
    return partials.sum() / BATCH


# speedup vs baseline: 1.0230x; 1.0230x over previous
import functools

import jax
import jax.numpy as jnp
from jax import lax
from jax.experimental import pallas as pl
from jax.experimental.pallas import tpu as pltpu
from jax.experimental.pallas import tpu_sc as plsc

VOCAB = 1000000
EMBED = 64
CTX = 20
BATCH = 16384

NW = 32
PER_W = BATCH // NW
CH = 8
NCHUNK = PER_W // CH
CROWS = CH * CTX
L = 16
W128 = 2 * EMBED
CPW = PER_W * CTX


def _sc_body(b_hbm, cf_hbm, nb_hbm, ncf_hbm, emb_hbm, ctx_hbm, simi_hbm,
             out_hbm, braw, nbraw, craw, ncraw,
             bidx_a, nbidx_a, cidx_a, ncidx_a,
             bidx_b, nbidx_b, cidx_b, ncidx_b,
             be_a, nbe_a, ce_a, nce_a, be_b, nbe_b, ce_b, nce_b,
             ctxv, simiv, outv, wbuf, sem_a, sem_b):
    cid = lax.axis_index("c")
    sid = lax.axis_index("s")
    wid = sid * 2 + cid
    base = wid * PER_W
    lanes = lax.iota(jnp.int32, L)

    ob = pl.multiple_of(base, PER_W)
    o20 = pl.multiple_of(base * CTX, CPW)
    pltpu.sync_copy(b_hbm.at[pl.ds(ob, PER_W)], braw.at[pl.ds(0, PER_W)])
    pltpu.sync_copy(nb_hbm.at[pl.ds(ob, PER_W)], nbraw.at[pl.ds(0, PER_W)])
    pltpu.sync_copy(cf_hbm.at[pl.ds(o20, CPW)], craw)
    pltpu.sync_copy(ncf_hbm.at[pl.ds(o20, CPW)], ncraw)
    braw[pl.ds(PER_W, L)] = jnp.zeros((L,), jnp.int32)
    nbraw[pl.ds(PER_W, L)] = jnp.zeros((L,), jnp.int32)

    pltpu.sync_copy(ctx_hbm, ctxv)
    x0 = ctxv[pl.ds(0, L)]
    x1 = ctxv[pl.ds(L, L)]

    _dnums = lax.GatherDimensionNumbers(
        offset_dims=(), collapsed_slice_dims=(0,), start_index_map=(0,))

    def _gat(x, idx):
        return lax.gather(x, idx[:, None], _dnums, (1,),
                          mode=lax.GatherScatterMode.PROMISE_IN_BOUNDS)

    def _rot(x, sh):
        return _gat(x, (lanes + sh) & (L - 1))

    def _splat(x, j):
        return _gat(x, jnp.full((L,), j, jnp.int32))

    m = jnp.maximum(x0, x1)
    for sh in (8, 4, 2, 1):
        m = jnp.maximum(m, _rot(m, sh))
    e0 = jnp.exp(x0 - m)
    e1 = jnp.exp(x1 - m)
    s = e0 + e1
    for sh in (8, 4, 2, 1):
        s = s + _rot(s, sh)
    inv_s = 1.0 / s
    for j in range(CTX):
        src = e0 if j < L else e1
        wbuf[pl.ds(j * L, L)] = _splat(src, j % L) * inv_s

    tl = [t * L + lanes for t in range(4)]

    bufs = ((bidx_a, nbidx_a, cidx_a, ncidx_a, be_a, nbe_a, ce_a, nce_a,
             sem_a),
            (bidx_b, nbidx_b, cidx_b, ncidx_b, be_b, nbe_b, ce_b, nce_b,
             sem_b))

    def issue(c, which):
        bidx, nbidx, cidx, ncidx, be, nbe, ce, nce, sem = bufs[which]
        coff = c * CROWS
        boff = c * CH
        for g in range(CTX // 2):
            gs = pl.ds(g * L, L)
            cidx[gs] = craw[pl.ds(coff + g * L, L)] >> 1
            ncidx[gs] = ncraw[pl.ds(coff + g * L, L)] >> 1
        bv = braw[pl.ds(boff, L)] >> 1
        nbv = nbraw[pl.ds(boff, L)] >> 1
        bidx[...] = bv
        nbidx[...] = nbv
        for lo, n in ((0, 128), (128, 32)):
            pltpu.async_copy(
                emb_hbm.at[cidx.at[pl.ds(lo, n)]], ce.at[pl.ds(lo, n)], sem)
            pltpu.async_copy(
                emb_hbm.at[ncidx.at[pl.ds(lo, n)]], nce.at[pl.ds(lo, n)],
                sem)
        pltpu.async_copy(emb_hbm.at[bidx.at[pl.ds(0, CH)]], be, sem)
        pltpu.async_copy(emb_hbm.at[nbidx.at[pl.ds(0, CH)]], nbe, sem)

    def drain(which):
        bidx, nbidx, cidx, ncidx, be, nbe, ce, nce, sem = bufs[which]
        pltpu.make_async_copy(emb_hbm.at[pl.ds(0, CROWS)], ce, sem).wait()
        pltpu.make_async_copy(emb_hbm.at[pl.ds(0, CROWS)], nce, sem).wait()
        pltpu.make_async_copy(emb_hbm.at[pl.ds(0, CH)], be, sem).wait()
        pltpu.make_async_copy(emb_hbm.at[pl.ds(0, CH)], nbe, sem).wait()

    def compute(c, which, acc):
        _, _, _, _, be, nbe, ce, nce, _ = bufs[which]
        coff = c * CROWS
        boff = c * CH
        bpo = (braw[pl.ds(boff, L)] & 1) * EMBED
        nbpo = (nbraw[pl.ds(boff, L)] & 1) * EMBED

        def elem(i, el_acc):
            a0, a1, a2, a3 = el_acc
            r0 = i * CTX
            iv = jnp.full((L,), i, jnp.int32)
            bcol = _gat(bpo, iv)
            nbcol = _gat(nbpo, iv)
            bs = [plsc.load_gather(be, [iv, bcol + tl[t]]) for t in range(4)]
            ns = [plsc.load_gather(nbe, [iv, nbcol + tl[t]])
                  for t in range(4)]
            u = [0.5 * ns[t] - bs[t] for t in range(4)]
            v = [0.5 * bs[t] for t in range(4)]
            ra = craw[pl.ds(coff + r0, L)]
            rb = craw[pl.ds(coff + r0 + 4, L)]
            nra = ncraw[pl.ds(coff + r0, L)]
            nrb = ncraw[pl.ds(coff + r0 + 4, L)]
            fba = (ra & 1) * EMBED
            fbb = (rb & 1) * EMBED
            nfba = (nra & 1) * EMBED
            nfbb = (nrb & 1) * EMBED
            for j in range(CTX):
                r = r0 + j
                rrow = jnp.full((L,), r, jnp.int32)
                if j < L:
                    ccol = _splat(fba, j)
                    nccol = _splat(nfba, j)
                else:
                    ccol = _splat(fbb, j - 4)
                    nccol = _splat(nfbb, j - 4)
                cs = [plsc.load_gather(ce, [rrow, ccol + tl[t]])
                      for t in range(4)]
                nc = [plsc.load_gather(nce, [rrow, nccol + tl[t]])
                      for t in range(4)]
                t0 = ((u[0] * cs[0] + u[1] * cs[1])
                      + (u[2] * cs[2] + u[3] * cs[3]))
                t1 = ((v[0] * nc[0] + v[1] * nc[1])
                      + (v[2] * nc[2] + v[3] * nc[3]))
                wv = wbuf[pl.ds(j * L, L)]
                term = wv * (t0 + t1)
                if j % 4 == 0:
                    a0 = a0 + term
                elif j % 4 == 1:
                    a1 = a1 + term
                elif j % 4 == 2:
                    a2 = a2 + term
                else:
                    a3 = a3 + term
            return (a0, a1, a2, a3)

        return lax.fori_loop(0, CH, elem, acc)

    issue(0, 0)
    issue(1, 1)

    def pair(mm, acc):
        c0 = mm * 2
        drain(0)
        acc = compute(c0, 0, acc)

        @pl.when(mm < NCHUNK // 2 - 1)
        def _():
            issue(c0 + 2, 0)

        drain(1)
        acc = compute(c0 + 1, 1, acc)

        @pl.when(mm < NCHUNK // 2 - 1)
        def _():
            issue(c0 + 3, 1)

        return acc

    z = jnp.zeros((L,), jnp.float32)
    a0, a1, a2, a3 = lax.fori_loop(0, NCHUNK // 2, pair, (z, z, z, z))
    outv[...] = (a0 + a1) + (a2 + a3)

    @pl.when(wid == 0)
    def _():
        pltpu.sync_copy(simi_hbm, simiv)

        def sbody(k, a2s):
            off = pl.multiple_of(k * L, L)
            vv = simiv[pl.ds(off, L)]
            return a2s + vv * vv

        a2s = lax.fori_loop(0, (EMBED * EMBED) // L, sbody,
                            jnp.zeros((L,), jnp.float32))
        outv[...] = outv[...] + (0.1 * BATCH / (EMBED * EMBED)) * a2s

    oo = pl.multiple_of(wid * L, L)
    pltpu.sync_copy(outv, out_hbm.at[pl.ds(oo, L)])


@functools.partial(
    pl.kernel,
    out_type=jax.ShapeDtypeStruct((NW * L,), jnp.float32),
    mesh=plsc.VectorSubcoreMesh(core_axis_name="c", subcore_axis_name="s"),
    compiler_params=pltpu.CompilerParams(needs_layout_passes=False),
    scratch_types=[
        pltpu.VMEM((PER_W + L,), jnp.int32),
        pltpu.VMEM((PER_W + L,), jnp.int32),
        pltpu.VMEM((CPW,), jnp.int32),
        pltpu.VMEM((CPW,), jnp.int32),
        pltpu.VMEM((L,), jnp.int32),
        pltpu.VMEM((L,), jnp.int32),
        pltpu.VMEM((CROWS,), jnp.int32),
        pltpu.VMEM((CROWS,), jnp.int32),
        pltpu.VMEM((L,), jnp.int32),
        pltpu.VMEM((L,), jnp.int32),
        pltpu.VMEM((CROWS,), jnp.int32),
        pltpu.VMEM((CROWS,), jnp.int32),
        pltpu.VMEM((CH, W128), jnp.float32),
        pltpu.VMEM((CH, W128), jnp.float32),
        pltpu.VMEM((CROWS, W128), jnp.float32),
        pltpu.VMEM((CROWS, W128), jnp.float32),
        pltpu.VMEM((CH, W128), jnp.float32),
        pltpu.VMEM((CH, W128), jnp.float32),
        pltpu.VMEM((CROWS, W128), jnp.float32),
        pltpu.VMEM((CROWS, W128), jnp.float32),
        pltpu.VMEM((32,), jnp.float32),
        pltpu.VMEM((EMBED * EMBED,), jnp.float32),
        pltpu.VMEM((L,), jnp.float32),
        pltpu.VMEM((CTX * L,), jnp.float32),
        pltpu.SemaphoreType.DMA,
        pltpu.SemaphoreType.DMA,
    ],
)
def _sc_entry(*refs):
    _sc_body(*refs)


def kernel(b, C, nb, nC, emb_weight, ctx_scheme, simi_kernel):
    b32 = b.astype(jnp.int32)
    nb32 = nb.astype(jnp.int32)
    cf = C.reshape(-1).astype(jnp.int32)
    ncf = nC.reshape(-1).astype(jnp.int32)
    emb2 = emb_weight.reshape(VOCAB // 2, W128)
    ctx_pad = jnp.concatenate(
        [ctx_scheme.astype(jnp.float32),
         jnp.full((32 - CTX,), -1e30, jnp.float32)])
    partials = _sc_entry(b32, cf, nb32, ncf, emb2, ctx_pad,
                         simi_kernel.reshape(-1))
    return partials.sum() / BATCH
